# baseline (speedup 1.0000x reference)
import functools

import jax
import jax.numpy as jnp
from jax import lax
from jax.experimental import pallas as pl
from jax.experimental.pallas import tpu as pltpu
from jax.experimental.pallas import tpu_sc as plsc

N = 10000
E = 160000
D_IN = 256
D_H = 512
D_OUT = 256

NC = 2
NS = 16
L = 16
HALF = 128
EPT = E // NS
K = 64
CHUNKS = EPT // K
TAIL = EPT - CHUNKS * K
IB = 16
STRIPE = N // NS
STRIPE_AL = 632


def _spmm_body(table, src, dst, w, init, out, acc, srcb,
               r0, r1, sb0, sb1, w0, w1, d0, d1, d2, d3, dtail, initbuf,
               sg0, sg1, ss0, ss1, sw0, sw1, sd0, sd1, sd2, sd3, sem):
  c = lax.axis_index("c")
  s = lax.axis_index("s")
  c_n = c * N
  start = pl.multiple_of((s * STRIPE) // 8 * 8, 8)
  base = s * EPT
  rbufs = (r0, r1)
  sbufs = (sb0, sb1)
  wbufs = (w0, w1)
  dbufs = (d0, d1, d2, d3)
  sgs = (sg0, sg1)
  sss = (ss0, ss1)
  sws = (sw0, sw1)
  sds = (sd0, sd1, sd2, sd3)

  h_src = pltpu.async_copy(src.at[pl.ds(base, EPT)], srcb, sem)

  pltpu.sync_copy(init.at[c], initbuf.at[0])
  bias_regs = [initbuf[0, pl.ds(j * L, L)] for j in range(HALF // L)]

  def fill_body(r, fcarry):
    for j in range(HALF // L):
      initbuf[r, pl.ds(j * L, L)] = bias_regs[j]
    return fcarry

  lax.fori_loop(1, IB, fill_body, 0)
  for q in range(STRIPE_AL // IB):
    pltpu.sync_copy(initbuf, acc.at[pl.ds(start + q * IB, IB)])
  rem = STRIPE_AL - (STRIPE_AL // IB) * IB
  if rem:
    pltpu.sync_copy(initbuf.at[pl.ds(0, rem)],
                    acc.at[pl.ds(start + STRIPE_AL - rem, rem)])
  h_src.wait()

  def adj_body(t, acarry):
    sl = pl.ds(t * L, L)
    srcb[sl] = srcb[sl] + c_n
    return acarry

  lax.fori_loop(0, EPT // L, adj_body, 0)
  plsc.subcore_barrier()

  def gather_start(i, rb, semb):
    pltpu.async_copy(table.at[srcb.at[pl.ds(i * K, K)]], rb, semb)

  def gather_wait(rb, semb):
    pltpu.make_async_copy(table.at[srcb.at[pl.ds(0, K)]], rb, semb).wait()

  def d_start(i, db, semb):
    pltpu.async_copy(dst.at[pl.ds(base + i * K, K)], db, semb)

  def d_wait(db, semb):
    pltpu.make_async_copy(dst.at[pl.ds(0, K)], db, semb).wait()

  def w_start(i, wv, semb):
    pltpu.async_copy(w.at[pl.ds(base + i * K, K)], wv, semb)

  def w_wait(wv, semb):
    pltpu.make_async_copy(w.at[pl.ds(0, K)], wv, semb).wait()

  def scale(rb, wv, sb):
    def edge_body(g, ecarry):
      w16 = wv[pl.ds(g * L, L)]
      for l in range(L):
        wspl = jnp.broadcast_to(w16[l], (L,))
        e = g * L + l
        for j in range(HALF // L):
          sl = pl.ds(j * L, L)
          sb[e, sl] = rb[e, sl] * wspl
      return ecarry

    lax.fori_loop(0, K // L, edge_body, 0)

  for i in range(2):
    gather_start(i, rbufs[i], sgs[i])
    w_start(i, wbufs[i], sws[i])
    d_start(i, dbufs[i], sds[i])

  def chunk(q, k):
    i = 4 * q + k
    b = k % 2
    dn = (k + 2) % 4
    if k in (0, 1):
      @pl.when(q > 0)
      def _():
        pltpu.make_async_copy(sbufs[b], acc.at[dbufs[dn]], sss[b]).wait()

      d_start(i + 2, dbufs[dn], sds[dn])
    else:
      pltpu.make_async_copy(sbufs[b], acc.at[dbufs[dn]], sss[b]).wait()

      @pl.when(q < CHUNKS // 4 - 1)
      def _():
        d_start(i + 2, dbufs[dn], sds[dn])

    gather_wait(rbufs[b], sgs[b])
    w_wait(wbufs[b], sws[b])
    scale(rbufs[b], wbufs[b], sbufs[b])
    if k in (0, 1):
      gather_start(i + 2, rbufs[b], sgs[b])
      w_start(i + 2, wbufs[b], sws[b])
    else:
      @pl.when(q < CHUNKS // 4 - 1)
      def _():
        gather_start(i + 2, rbufs[b], sgs[b])
        w_start(i + 2, wbufs[b], sws[b])

    d_wait(dbufs[k], sds[k])
    pltpu.async_copy(sbufs[b], acc.at[dbufs[k]], sss[b], add=True)

  def quad_body(q, carry):
    for k in range(4):
      chunk(q, k)
    return carry

  lax.fori_loop(0, CHUNKS // 4, quad_body, 0)
  for b in range(2):
    pltpu.make_async_copy(sbufs[b], acc.at[dbufs[2 + b]], sss[b]).wait()

  tb = CHUNKS * K
  pltpu.async_copy(table.at[srcb.at[pl.ds(tb, TAIL)]],
                   r0.at[pl.ds(0, TAIL)], sg0).wait()
  pltpu.sync_copy(w.at[pl.ds(base + tb, TAIL)], w0.at[pl.ds(0, TAIL)])
  w16 = w0[pl.ds(0, L)]
  for l in range(L):
    wspl = jnp.broadcast_to(w16[l], (L,))
    for j in range(HALF // L):
      sl = pl.ds(j * L, L)
      r0[l, sl] = r0[l, sl] * wspl
  pltpu.sync_copy(dst.at[pl.ds(base + tb, TAIL)], dtail)
  pltpu.sync_copy(r0.at[pl.ds(0, TAIL)], acc.at[dtail], add=True)
  plsc.subcore_barrier()

  pltpu.sync_copy(acc.at[pl.ds(start, STRIPE_AL)],
                  out.at[pl.ds(pl.multiple_of(c_n + start, 8), STRIPE_AL)])


_spmm = functools.partial(
    pl.kernel,
    out_type=jax.ShapeDtypeStruct((2 * N, HALF), jnp.float32),
    mesh=plsc.VectorSubcoreMesh(core_axis_name="c", subcore_axis_name="s"),
    scratch_types=[
        pltpu.VMEM_SHARED((N, HALF), jnp.float32),
        pltpu.VMEM((EPT,), jnp.int32),
        pltpu.VMEM((K, HALF), jnp.float32),
        pltpu.VMEM((K, HALF), jnp.float32),
        pltpu.VMEM((K, HALF), jnp.float32),
        pltpu.VMEM((K, HALF), jnp.float32),
        pltpu.VMEM((K,), jnp.float32),
        pltpu.VMEM((K,), jnp.float32),
        pltpu.VMEM((K,), jnp.int32),
        pltpu.VMEM((K,), jnp.int32),
        pltpu.VMEM((K,), jnp.int32),
        pltpu.VMEM((K,), jnp.int32),
        pltpu.VMEM((TAIL,), jnp.int32),
        pltpu.VMEM((IB, HALF), jnp.float32),
        pltpu.SemaphoreType.DMA,
        pltpu.SemaphoreType.DMA,
        pltpu.SemaphoreType.DMA,
        pltpu.SemaphoreType.DMA,
        pltpu.SemaphoreType.DMA,
        pltpu.SemaphoreType.DMA,
        pltpu.SemaphoreType.DMA,
        pltpu.SemaphoreType.DMA,
        pltpu.SemaphoreType.DMA,
        pltpu.SemaphoreType.DMA,
        pltpu.SemaphoreType.DMA,
    ],
)(_spmm_body)


BN = 1000


def _dense_body(s1_ref, w1_ref, b1_ref, w2_ref, out_ref):
  h = lax.dot_general(s1_ref[0], w1_ref[:, :HALF], (((1,), (1,)), ((), ())),
                      preferred_element_type=jnp.float32)
  h += lax.dot_general(s1_ref[1], w1_ref[:, HALF:], (((1,), (1,)), ((), ())),
                       preferred_element_type=jnp.float32)
  h = jnp.maximum(h + b1_ref[...], 0.0)
  g = lax.dot_general(h, w2_ref[...], (((1,), (1,)), ((), ())),
                      preferred_element_type=jnp.float32)
  out_ref[0] = g[:, :HALF]
  out_ref[1] = g[:, HALF:]


def _dense(s1, w1, b1, w2):
  return pl.pallas_call(
      _dense_body,
      grid=(N // BN,),
      in_specs=[
          pl.BlockSpec((2, BN, HALF), lambda i: (0, i, 0)),
          pl.BlockSpec((D_H, D_IN), lambda i: (0, 0)),
          pl.BlockSpec((1, D_H), lambda i: (0, 0)),
          pl.BlockSpec((D_OUT, D_H), lambda i: (0, 0)),
      ],
      out_specs=pl.BlockSpec((2, BN, HALF), lambda i: (0, i, 0)),
      out_shape=jax.ShapeDtypeStruct((2, N, HALF), jnp.float32),
  )(s1, w1, b1, w2)


def kernel(x, y, mask, edge_index, edge_weight, W1, b1, W2, b2):
  src = edge_index[0]
  dst = edge_index[1]
  x_sc = jnp.concatenate([x[:, :HALF], x[:, HALF:]], axis=0)
  zinit = jnp.zeros((2, HALF), dtype=jnp.float32)
  s1 = _spmm(x_sc, src, dst, edge_weight, zinit)
  g = _dense(s1.reshape(2, N, HALF), W1, b1.reshape(1, D_H), W2)
  b2init = jnp.stack([b2[:HALF], b2[HALF:]])
  o2 = _spmm(g.reshape(2 * N, HALF), src, dst, edge_weight, b2init)
  out = jnp.concatenate([o2[:N], o2[N:]], axis=1)
  return out, y


# speedup vs baseline: 1.2424x; 1.2424x over previous
import functools

import jax
import jax.numpy as jnp
from jax import lax
from jax.experimental import pallas as pl
from jax.experimental.pallas import tpu as pltpu
from jax.experimental.pallas import tpu_sc as plsc

N = 10000
E = 160000
D_IN = 256
D_H = 512
D_OUT = 256

NC = 2
NS = 16
L = 16
HALF = 128
EPT = E // NS
K = 64
CHUNKS = EPT // K
TAIL = EPT - CHUNKS * K
IB = 16
STRIPE = N // NS
STRIPE_AL = 632


def _spmm_body(table, src, dst, w, init, out, acc, srcb, dstb, wb,
               r0, r1, d0, d1, dtail, initbuf,
               sg0, sg1, ss0, ss1, sem):
  c = lax.axis_index("c")
  s = lax.axis_index("s")
  c_n = c * N
  start = pl.multiple_of((s * STRIPE) // 8 * 8, 8)
  base = s * EPT

  h_src = pltpu.async_copy(src.at[pl.ds(base, EPT)], srcb, sem)
  h_dst = pltpu.async_copy(dst.at[pl.ds(base, EPT)], dstb, sem)
  h_w = pltpu.async_copy(w.at[pl.ds(base, EPT)], wb, sem)

  pltpu.sync_copy(init.at[c], initbuf.at[0])
  bias_regs = [initbuf[0, pl.ds(j * L, L)] for j in range(HALF // L)]

  def fill_body(r, fcarry):
    for j in range(HALF // L):
      initbuf[r, pl.ds(j * L, L)] = bias_regs[j]
    return fcarry

  lax.fori_loop(1, IB, fill_body, 0)
  for q in range(STRIPE_AL // IB):
    pltpu.sync_copy(initbuf, acc.at[pl.ds(start + q * IB, IB)])
  rem = STRIPE_AL - (STRIPE_AL // IB) * IB
  if rem:
    pltpu.sync_copy(initbuf.at[pl.ds(0, rem)],
                    acc.at[pl.ds(start + STRIPE_AL - rem, rem)])
  h_src.wait()
  h_dst.wait()
  h_w.wait()

  def adj_body(t, acarry):
    sl = pl.ds(t * L, L)
    srcb[sl] = srcb[sl] + c_n
    return acarry

  lax.fori_loop(0, EPT // L, adj_body, 0)
  plsc.subcore_barrier()

  def gather_start(i, rb, semb):
    pltpu.async_copy(table.at[srcb.at[pl.ds(i * K, K)]], rb, semb)

  def gather_wait(rb, semb):
    pltpu.make_async_copy(table.at[srcb.at[pl.ds(0, K)]], rb, semb).wait()

  def scale(i, rb):
    def edge_body(g, ecarry):
      w16 = wb[pl.ds(i * K + g * L, L)]
      for l in range(L):
        wspl = jnp.broadcast_to(w16[l], (L,))
        e = g * L + l
        for j in range(HALF // L):
          sl = pl.ds(j * L, L)
          rb[e, sl] = rb[e, sl] * wspl
      return ecarry

    lax.fori_loop(0, K // L, edge_body, 0)

  def dcopy(i, db):
    for j in range(K // L):
      db[pl.ds(j * L, L)] = dstb[pl.ds(i * K + j * L, L)]

  gather_start(0, r0, sg0)

  def pair_body(t, carry):
    i0 = 2 * t
    gather_wait(r0, sg0)

    @pl.when(t > 0)
    def _():
      pltpu.make_async_copy(r1, acc.at[d1], ss1).wait()

    gather_start(i0 + 1, r1, sg1)
    scale(i0, r0)
    dcopy(i0, d0)
    pltpu.async_copy(r0, acc.at[d0], ss0, add=True)
    gather_wait(r1, sg1)
    pltpu.make_async_copy(r0, acc.at[d0], ss0).wait()

    @pl.when(i0 + 2 < CHUNKS)
    def _():
      gather_start(i0 + 2, r0, sg0)

    scale(i0 + 1, r1)
    dcopy(i0 + 1, d1)
    pltpu.async_copy(r1, acc.at[d1], ss1, add=True)
    return carry

  lax.fori_loop(0, CHUNKS // 2, pair_body, 0)
  pltpu.make_async_copy(r1, acc.at[d1], ss1).wait()

  tb = CHUNKS * K
  pltpu.async_copy(table.at[srcb.at[pl.ds(tb, TAIL)]],
                   r0.at[pl.ds(0, TAIL)], sg0).wait()
  w16 = wb[pl.ds(tb, L)]
  for l in range(L):
    wspl = jnp.broadcast_to(w16[l], (L,))
    for j in range(HALF // L):
      sl = pl.ds(j * L, L)
      r0[l, sl] = r0[l, sl] * wspl
  dtail[...] = dstb[pl.ds(tb, TAIL)]
  pltpu.sync_copy(r0.at[pl.ds(0, TAIL)], acc.at[dtail], add=True)
  plsc.subcore_barrier()

  pltpu.sync_copy(acc.at[pl.ds(start, STRIPE_AL)],
                  out.at[pl.ds(pl.multiple_of(c_n + start, 8), STRIPE_AL)])


_spmm = functools.partial(
    pl.kernel,
    out_type=jax.ShapeDtypeStruct((2 * N, HALF), jnp.float32),
    mesh=plsc.VectorSubcoreMesh(core_axis_name="c", subcore_axis_name="s"),
    scratch_types=[
        pltpu.VMEM_SHARED((N, HALF), jnp.float32),
        pltpu.VMEM((EPT,), jnp.int32),
        pltpu.VMEM((EPT,), jnp.int32),
        pltpu.VMEM((EPT,), jnp.float32),
        pltpu.VMEM((K, HALF), jnp.float32),
        pltpu.VMEM((K, HALF), jnp.float32),
        pltpu.VMEM((K,), jnp.int32),
        pltpu.VMEM((K,), jnp.int32),
        pltpu.VMEM((TAIL,), jnp.int32),
        pltpu.VMEM((IB, HALF), jnp.float32),
        pltpu.SemaphoreType.DMA,
        pltpu.SemaphoreType.DMA,
        pltpu.SemaphoreType.DMA,
        pltpu.SemaphoreType.DMA,
        pltpu.SemaphoreType.DMA,
    ],
)(_spmm_body)


BN = 1000


def _dense_body(s1_ref, w1_ref, b1_ref, w2_ref, out_ref):
  h = lax.dot_general(s1_ref[0], w1_ref[:, :HALF], (((1,), (1,)), ((), ())),
                      preferred_element_type=jnp.float32)
  h += lax.dot_general(s1_ref[1], w1_ref[:, HALF:], (((1,), (1,)), ((), ())),
                       preferred_element_type=jnp.float32)
  h = jnp.maximum(h + b1_ref[...], 0.0)
  g = lax.dot_general(h, w2_ref[...], (((1,), (1,)), ((), ())),
                      preferred_element_type=jnp.float32)
  out_ref[0] = g[:, :HALF]
  out_ref[1] = g[:, HALF:]


def _dense(s1, w1, b1, w2):
  return pl.pallas_call(
      _dense_body,
      grid=(N // BN,),
      in_specs=[
          pl.BlockSpec((2, BN, HALF), lambda i: (0, i, 0)),
          pl.BlockSpec((D_H, D_IN), lambda i: (0, 0)),
          pl.BlockSpec((1, D_H), lambda i: (0, 0)),
          pl.BlockSpec((D_OUT, D_H), lambda i: (0, 0)),
      ],
      out_specs=pl.BlockSpec((2, BN, HALF), lambda i: (0, i, 0)),
      out_shape=jax.ShapeDtypeStruct((2, N, HALF), jnp.float32),
  )(s1, w1, b1, w2)


def kernel(x, y, mask, edge_index, edge_weight, W1, b1, W2, b2):
  src = edge_index[0]
  dst = edge_index[1]
  x_sc = jnp.concatenate([x[:, :HALF], x[:, HALF:]], axis=0)
  zinit = jnp.zeros((2, HALF), dtype=jnp.float32)
  s1 = _spmm(x_sc, src, dst, edge_weight, zinit)
  g = _dense(s1.reshape(2, N, HALF), W1, b1.reshape(1, D_H), W2)
  b2init = jnp.stack([b2[:HALF], b2[HALF:]])
  o2 = _spmm(g.reshape(2 * N, HALF), src, dst, edge_weight, b2init)
  out = jnp.concatenate([o2[:N], o2[N:]], axis=1)
  return out, y


# speedup vs baseline: 1.4576x; 1.1732x over previous
import functools

import jax
import jax.numpy as jnp
from jax import lax
from jax.experimental import pallas as pl
from jax.experimental.pallas import tpu as pltpu
from jax.experimental.pallas import tpu_sc as plsc

N = 10000
E = 160000
D_IN = 256
D_H = 512
D_OUT = 256

NC = 2
NS = 16
L = 16
HALF = 128
EPT = E // NS
K = 96
CHUNKS = EPT // K
TAIL = EPT - CHUNKS * K
IB = 16
STRIPE = N // NS
STRIPE_AL = 632


def _spmm_body(table, src, dst, w, init, out, acc, srcb, wb,
               r0, r1, d0, d1, dtail, initbuf,
               sg0, sg1, ss0, ss1, sd0, sd1, sem):
  c = lax.axis_index("c")
  s = lax.axis_index("s")
  c_n = c * N
  start = pl.multiple_of((s * STRIPE) // 8 * 8, 8)
  base = s * EPT

  h_src = pltpu.async_copy(src.at[pl.ds(base, EPT)], srcb, sem)
  h_w = pltpu.async_copy(w.at[pl.ds(base, EPT)], wb, sem)

  pltpu.sync_copy(init.at[c], initbuf.at[0])
  bias_regs = [initbuf[0, pl.ds(j * L, L)] for j in range(HALF // L)]

  def fill_body(r, fcarry):
    for j in range(HALF // L):
      initbuf[r, pl.ds(j * L, L)] = bias_regs[j]
    return fcarry

  lax.fori_loop(1, IB, fill_body, 0)
  for q in range(STRIPE_AL // IB):
    pltpu.sync_copy(initbuf, acc.at[pl.ds(start + q * IB, IB)])
  rem = STRIPE_AL - (STRIPE_AL // IB) * IB
  if rem:
    pltpu.sync_copy(initbuf.at[pl.ds(0, rem)],
                    acc.at[pl.ds(start + STRIPE_AL - rem, rem)])
  h_src.wait()
  h_w.wait()

  def adj_body(t, acarry):
    sl = pl.ds(t * L, L)
    srcb[sl] = srcb[sl] + c_n
    return acarry

  lax.fori_loop(0, EPT // L, adj_body, 0)
  plsc.subcore_barrier()

  def gather_start(i, rb, semb):
    pltpu.async_copy(table.at[srcb.at[pl.ds(i * K, K)]], rb, semb)

  def gather_wait(rb, semb):
    pltpu.make_async_copy(table.at[srcb.at[pl.ds(0, K)]], rb, semb).wait()

  def scale(i, rb):
    def edge_body(g, ecarry):
      w16 = wb[pl.ds(i * K + g * L, L)]
      for l in range(L):
        wspl = jnp.broadcast_to(w16[l], (L,))
        e = g * L + l
        for j in range(HALF // L):
          sl = pl.ds(j * L, L)
          rb[e, sl] = rb[e, sl] * wspl
      return ecarry

    lax.fori_loop(0, K // L, edge_body, 0)

  def d_start(i, db, semb):
    pltpu.async_copy(dst.at[pl.ds(base + i * K, K)], db, semb)

  def d_wait(db, semb):
    pltpu.make_async_copy(dst.at[pl.ds(0, K)], db, semb).wait()

  gather_start(0, r0, sg0)
  d_start(0, d0, sd0)

  def pair_body(t, carry):
    i0 = 2 * t
    gather_wait(r0, sg0)

    @pl.when(t > 0)
    def _():
      pltpu.make_async_copy(r1, acc.at[d1], ss1).wait()

    gather_start(i0 + 1, r1, sg1)
    d_start(i0 + 1, d1, sd1)
    scale(i0, r0)
    d_wait(d0, sd0)
    pltpu.async_copy(r0, acc.at[d0], ss0, add=True)
    gather_wait(r1, sg1)
    pltpu.make_async_copy(r0, acc.at[d0], ss0).wait()

    @pl.when(i0 + 2 < CHUNKS)
    def _():
      gather_start(i0 + 2, r0, sg0)
      d_start(i0 + 2, d0, sd0)

    scale(i0 + 1, r1)
    d_wait(d1, sd1)
    pltpu.async_copy(r1, acc.at[d1], ss1, add=True)
    return carry

  lax.fori_loop(0, CHUNKS // 2, pair_body, 0)
  pltpu.make_async_copy(r1, acc.at[d1], ss1).wait()

  tb = CHUNKS * K
  pltpu.async_copy(table.at[srcb.at[pl.ds(tb, TAIL)]],
                   r0.at[pl.ds(0, TAIL)], sg0).wait()
  w16 = wb[pl.ds(tb, L)]
  for l in range(L):
    wspl = jnp.broadcast_to(w16[l], (L,))
    for j in range(HALF // L):
      sl = pl.ds(j * L, L)
      r0[l, sl] = r0[l, sl] * wspl
  pltpu.sync_copy(dst.at[pl.ds(base + tb, TAIL)], dtail)
  pltpu.sync_copy(r0.at[pl.ds(0, TAIL)], acc.at[dtail], add=True)
  plsc.subcore_barrier()

  pltpu.sync_copy(acc.at[pl.ds(start, STRIPE_AL)],
                  out.at[pl.ds(pl.multiple_of(c_n + start, 8), STRIPE_AL)])


_spmm = functools.partial(
    pl.kernel,
    out_type=jax.ShapeDtypeStruct((2 * N, HALF), jnp.float32),
    mesh=plsc.VectorSubcoreMesh(core_axis_name="c", subcore_axis_name="s"),
    scratch_types=[
        pltpu.VMEM_SHARED((N, HALF), jnp.float32),
        pltpu.VMEM((EPT,), jnp.int32),
        pltpu.VMEM((EPT,), jnp.float32),
        pltpu.VMEM((K, HALF), jnp.float32),
        pltpu.VMEM((K, HALF), jnp.float32),
        pltpu.VMEM((K,), jnp.int32),
        pltpu.VMEM((K,), jnp.int32),
        pltpu.VMEM((TAIL,), jnp.int32),
        pltpu.VMEM((IB, HALF), jnp.float32),
        pltpu.SemaphoreType.DMA,
        pltpu.SemaphoreType.DMA,
        pltpu.SemaphoreType.DMA,
        pltpu.SemaphoreType.DMA,
        pltpu.SemaphoreType.DMA,
        pltpu.SemaphoreType.DMA,
        pltpu.SemaphoreType.DMA,
    ],
)(_spmm_body)


BN = 1000


def _dense_body(s1_ref, w1_ref, b1_ref, w2_ref, out_ref):
  h = lax.dot_general(s1_ref[0], w1_ref[:, :HALF], (((1,), (1,)), ((), ())),
                      preferred_element_type=jnp.float32)
  h += lax.dot_general(s1_ref[1], w1_ref[:, HALF:], (((1,), (1,)), ((), ())),
                       preferred_element_type=jnp.float32)
  h = jnp.maximum(h + b1_ref[...], 0.0)
  g = lax.dot_general(h, w2_ref[...], (((1,), (1,)), ((), ())),
                      preferred_element_type=jnp.float32)
  out_ref[0] = g[:, :HALF]
  out_ref[1] = g[:, HALF:]


def _dense(s1, w1, b1, w2):
  return pl.pallas_call(
      _dense_body,
      grid=(N // BN,),
      in_specs=[
          pl.BlockSpec((2, BN, HALF), lambda i: (0, i, 0)),
          pl.BlockSpec((D_H, D_IN), lambda i: (0, 0)),
          pl.BlockSpec((1, D_H), lambda i: (0, 0)),
          pl.BlockSpec((D_OUT, D_H), lambda i: (0, 0)),
      ],
      out_specs=pl.BlockSpec((2, BN, HALF), lambda i: (0, i, 0)),
      out_shape=jax.ShapeDtypeStruct((2, N, HALF), jnp.float32),
  )(s1, w1, b1, w2)


def kernel(x, y, mask, edge_index, edge_weight, W1, b1, W2, b2):
  src = edge_index[0]
  dst = edge_index[1]
  x_sc = jnp.concatenate([x[:, :HALF], x[:, HALF:]], axis=0)
  zinit = jnp.zeros((2, HALF), dtype=jnp.float32)
  s1 = _spmm(x_sc, src, dst, edge_weight, zinit)
  g = _dense(s1.reshape(2, N, HALF), W1, b1.reshape(1, D_H), W2)
  b2init = jnp.stack([b2[:HALF], b2[HALF:]])
  o2 = _spmm(g.reshape(2 * N, HALF), src, dst, edge_weight, b2init)
  out = jnp.concatenate([o2[:N], o2[N:]], axis=1)
  return out, y


# speedup vs baseline: 1.5368x; 1.0544x over previous
import functools

import jax
import jax.numpy as jnp
from jax import lax
from jax.experimental import pallas as pl
from jax.experimental.pallas import tpu as pltpu
from jax.experimental.pallas import tpu_sc as plsc

N = 10000
E = 160000
D_IN = 256
D_H = 512
D_OUT = 256

NC = 2
NS = 16
L = 16
HALF = 128
EPT = E // NS
K = 128
CHUNKS = EPT // K
TAIL = EPT - CHUNKS * K
IB = 16
STRIPE = N // NS
STRIPE_AL = 632


def _spmm_body(table, src, dst, w, init, out, acc, srcb,
               r0, r1, w0, w1, d0, d1, dtail, initbuf,
               sg0, sg1, ss0, ss1, sw0, sw1, sd0, sd1, sem):
  c = lax.axis_index("c")
  s = lax.axis_index("s")
  c_n = c * N
  start = pl.multiple_of((s * STRIPE) // 8 * 8, 8)
  base = s * EPT

  h_src = pltpu.async_copy(src.at[pl.ds(base, EPT)], srcb, sem)

  pltpu.sync_copy(init.at[c], initbuf.at[0])
  bias_regs = [initbuf[0, pl.ds(j * L, L)] for j in range(HALF // L)]

  def fill_body(r, fcarry):
    for j in range(HALF // L):
      initbuf[r, pl.ds(j * L, L)] = bias_regs[j]
    return fcarry

  lax.fori_loop(1, IB, fill_body, 0)
  for q in range(STRIPE_AL // IB):
    pltpu.sync_copy(initbuf, acc.at[pl.ds(start + q * IB, IB)])
  rem = STRIPE_AL - (STRIPE_AL // IB) * IB
  if rem:
    pltpu.sync_copy(initbuf.at[pl.ds(0, rem)],
                    acc.at[pl.ds(start + STRIPE_AL - rem, rem)])
  h_src.wait()

  def adj_body(t, acarry):
    sl = pl.ds(t * L, L)
    srcb[sl] = srcb[sl] + c_n
    return acarry

  lax.fori_loop(0, EPT // L, adj_body, 0)
  plsc.subcore_barrier()

  def gather_start(i, rb, semb):
    pltpu.async_copy(table.at[srcb.at[pl.ds(i * K, K)]], rb, semb)

  def gather_wait(rb, semb):
    pltpu.make_async_copy(table.at[srcb.at[pl.ds(0, K)]], rb, semb).wait()

  def scale(rb, wv):
    def edge_body(g, ecarry):
      w16 = wv[pl.ds(g * L, L)]
      for l in range(L):
        wspl = jnp.broadcast_to(w16[l], (L,))
        e = g * L + l
        for j in range(HALF // L):
          sl = pl.ds(j * L, L)
          rb[e, sl] = rb[e, sl] * wspl
      return ecarry

    lax.fori_loop(0, K // L, edge_body, 0)

  def d_start(i, db, semb):
    pltpu.async_copy(dst.at[pl.ds(base + i * K, K)], db, semb)

  def d_wait(db, semb):
    pltpu.make_async_copy(dst.at[pl.ds(0, K)], db, semb).wait()

  def w_start(i, wv, semb):
    pltpu.async_copy(w.at[pl.ds(base + i * K, K)], wv, semb)

  def w_wait(wv, semb):
    pltpu.make_async_copy(w.at[pl.ds(0, K)], wv, semb).wait()

  gather_start(0, r0, sg0)
  d_start(0, d0, sd0)
  w_start(0, w0, sw0)

  def pair_body(t, carry):
    i0 = 2 * t
    gather_wait(r0, sg0)

    @pl.when(t > 0)
    def _():
      pltpu.make_async_copy(r1, acc.at[d1], ss1).wait()

    gather_start(i0 + 1, r1, sg1)
    d_start(i0 + 1, d1, sd1)
    w_start(i0 + 1, w1, sw1)
    w_wait(w0, sw0)
    scale(r0, w0)
    d_wait(d0, sd0)
    pltpu.async_copy(r0, acc.at[d0], ss0, add=True)
    gather_wait(r1, sg1)
    pltpu.make_async_copy(r0, acc.at[d0], ss0).wait()

    @pl.when(i0 + 2 < CHUNKS)
    def _():
      gather_start(i0 + 2, r0, sg0)
      d_start(i0 + 2, d0, sd0)
      w_start(i0 + 2, w0, sw0)

    w_wait(w1, sw1)
    scale(r1, w1)
    d_wait(d1, sd1)
    pltpu.async_copy(r1, acc.at[d1], ss1, add=True)
    return carry

  lax.fori_loop(0, CHUNKS // 2, pair_body, 0)
  pltpu.make_async_copy(r1, acc.at[d1], ss1).wait()

  tb = CHUNKS * K
  pltpu.async_copy(table.at[srcb.at[pl.ds(tb, TAIL)]],
                   r0.at[pl.ds(0, TAIL)], sg0).wait()
  pltpu.sync_copy(w.at[pl.ds(base + tb, TAIL)], w0.at[pl.ds(0, TAIL)])
  w16 = w0[pl.ds(0, L)]
  for l in range(L):
    wspl = jnp.broadcast_to(w16[l], (L,))
    for j in range(HALF // L):
      sl = pl.ds(j * L, L)
      r0[l, sl] = r0[l, sl] * wspl
  pltpu.sync_copy(dst.at[pl.ds(base + tb, TAIL)], dtail)
  pltpu.sync_copy(r0.at[pl.ds(0, TAIL)], acc.at[dtail], add=True)
  plsc.subcore_barrier()

  pltpu.sync_copy(acc.at[pl.ds(start, STRIPE_AL)],
                  out.at[pl.ds(pl.multiple_of(c_n + start, 8), STRIPE_AL)])


_spmm = functools.partial(
    pl.kernel,
    out_type=jax.ShapeDtypeStruct((2 * N, HALF), jnp.float32),
    mesh=plsc.VectorSubcoreMesh(core_axis_name="c", subcore_axis_name="s"),
    scratch_types=[
        pltpu.VMEM_SHARED((N, HALF), jnp.float32),
        pltpu.VMEM((EPT,), jnp.int32),
        pltpu.VMEM((K, HALF), jnp.float32),
        pltpu.VMEM((K, HALF), jnp.float32),
        pltpu.VMEM((K,), jnp.float32),
        pltpu.VMEM((K,), jnp.float32),
        pltpu.VMEM((K,), jnp.int32),
        pltpu.VMEM((K,), jnp.int32),
        pltpu.VMEM((TAIL,), jnp.int32),
        pltpu.VMEM((IB, HALF), jnp.float32),
        pltpu.SemaphoreType.DMA,
        pltpu.SemaphoreType.DMA,
        pltpu.SemaphoreType.DMA,
        pltpu.SemaphoreType.DMA,
        pltpu.SemaphoreType.DMA,
        pltpu.SemaphoreType.DMA,
        pltpu.SemaphoreType.DMA,
        pltpu.SemaphoreType.DMA,
        pltpu.SemaphoreType.DMA,
    ],
)(_spmm_body)


BN = 1000


def _dense_body(s1_ref, w1_ref, b1_ref, w2_ref, out_ref):
  h = lax.dot_general(s1_ref[0], w1_ref[:, :HALF], (((1,), (1,)), ((), ())),
                      preferred_element_type=jnp.float32)
  h += lax.dot_general(s1_ref[1], w1_ref[:, HALF:], (((1,), (1,)), ((), ())),
                       preferred_element_type=jnp.float32)
  h = jnp.maximum(h + b1_ref[...], 0.0)
  g = lax.dot_general(h, w2_ref[...], (((1,), (1,)), ((), ())),
                      preferred_element_type=jnp.float32)
  out_ref[0] = g[:, :HALF]
  out_ref[1] = g[:, HALF:]


def _dense(s1, w1, b1, w2):
  return pl.pallas_call(
      _dense_body,
      grid=(N // BN,),
      in_specs=[
          pl.BlockSpec((2, BN, HALF), lambda i: (0, i, 0)),
          pl.BlockSpec((D_H, D_IN), lambda i: (0, 0)),
          pl.BlockSpec((1, D_H), lambda i: (0, 0)),
          pl.BlockSpec((D_OUT, D_H), lambda i: (0, 0)),
      ],
      out_specs=pl.BlockSpec((2, BN, HALF), lambda i: (0, i, 0)),
      out_shape=jax.ShapeDtypeStruct((2, N, HALF), jnp.float32),
  )(s1, w1, b1, w2)


def kernel(x, y, mask, edge_index, edge_weight, W1, b1, W2, b2):
  src = edge_index[0]
  dst = edge_index[1]
  x_sc = jnp.concatenate([x[:, :HALF], x[:, HALF:]], axis=0)
  zinit = jnp.zeros((2, HALF), dtype=jnp.float32)
  s1 = _spmm(x_sc, src, dst, edge_weight, zinit)
  g = _dense(s1.reshape(2, N, HALF), W1, b1.reshape(1, D_H), W2)
  b2init = jnp.stack([b2[:HALF], b2[HALF:]])
  o2 = _spmm(g.reshape(2 * N, HALF), src, dst, edge_weight, b2init)
  out = jnp.concatenate([o2[:N], o2[N:]], axis=1)
  return out, y
